# baseline (speedup 1.0000x reference)
import functools

import jax
import jax.numpy as jnp
from jax.experimental import pallas as pl
from jax.experimental.pallas import tpu as pltpu

_N = 10000
_E = 320000
_D = 128
_DEPTH = 3
_NC = 102


def _final_body(x_ref, w_ref, b_ref, o_ref):
    z = jnp.dot(x_ref[...], w_ref[...], preferred_element_type=jnp.float32)
    z = jnp.maximum(z + b_ref[...], 0.0)
    lane = jax.lax.broadcasted_iota(jnp.int32, z.shape, 1)
    z = jnp.where(lane < _NC, z, -1e30)
    m = jnp.max(z, axis=1, keepdims=True)
    y = z - m
    lse = jnp.log(jnp.sum(jnp.exp(y), axis=1, keepdims=True))
    o_ref[...] = y - lse


def _final_conv(xcat, W, b):
    n, k = xcat.shape
    Wp = jnp.zeros((k, 128), jnp.float32).at[:, :_NC].set(W)
    bp = jnp.zeros((1, 128), jnp.float32).at[0, :_NC].set(b)
    blk = 1000
    out = pl.pallas_call(
        _final_body,
        grid=(n // blk,),
        in_specs=[
            pl.BlockSpec((blk, k), lambda i: (i, 0)),
            pl.BlockSpec((k, 128), lambda i: (0, 0)),
            pl.BlockSpec((1, 128), lambda i: (0, 0)),
        ],
        out_specs=pl.BlockSpec((blk, 128), lambda i: (i, 0)),
        out_shape=jax.ShapeDtypeStruct((n, 128), jnp.float32),
    )(xcat, Wp, bp)
    return out[:, :_NC]


def _conv(x, src, dst, valid, W, b, n):
    v = valid.astype(x.dtype)[:, None]
    msg = x[src] * v
    seg = jnp.where(valid, dst, n)
    agg = jax.ops.segment_sum(msg, seg, num_segments=n + 1)[:n]
    return jax.nn.relu((x + agg) @ W + b)


def _pool(x, src, dst, valid, p, n):
    k = n // 2
    score = (x @ p) / (jnp.linalg.norm(p) + 1e-8)
    vals, idx = jax.lax.top_k(score, k)
    gate = jnp.tanh(vals)
    x_new = x[idx] * gate[:, None]
    sel = jnp.zeros((n,), dtype=bool).at[idx].set(True)
    perm = jnp.zeros((n,), dtype=jnp.int32).at[idx].set(jnp.arange(k, dtype=jnp.int32))
    new_valid = valid & sel[src] & sel[dst]
    return x_new, perm[src], perm[dst], new_valid, idx, k


def kernel(x, edge_index, Re, Wd0, bd0, Wd1, bd1, Wd2, bd2, p0, p1, p2, Wb, bb,
           Wu0, bu0, Wu1, bu1, Wu2, bu2):
    src = edge_index[0]
    dst = edge_index[1]
    Wd = [Wd0, Wd1, Wd2]; bd = [bd0, bd1, bd2]; pp = [p0, p1, p2]
    Wu = [Wu0, Wu1, Wu2]; bu = [bu0, bu1, bu2]
    valid = jnp.ones((src.shape[0],), dtype=bool)
    n = x.shape[0]
    x_skips = []; edge_skips = []; indcs = []
    for i in range(_DEPTH):
        x = _conv(x, src, dst, valid, Wd[i], bd[i], n)
        x_skips.append(x)
        edge_skips.append((src, dst, valid, n))
        x, src, dst, valid, idx, n = _pool(x, src, dst, valid, pp[i], n)
        indcs.append(idx)
    re_col = jnp.broadcast_to(Re[0], (x.shape[0], 1)).astype(x.dtype)
    x = jnp.concatenate([x, re_col], axis=1)
    x = _conv(x, src, dst, valid, Wb, bb, n)
    for i in range(_DEPTH):
        up = _DEPTH - i - 1
        skip = x_skips[up]
        src, dst, valid, n = edge_skips[up]
        idx = indcs[up]
        x = jnp.zeros((n, x.shape[1]), dtype=x.dtype).at[idx].set(x)
        x = jnp.concatenate([x, skip], axis=-1)
        if i < _DEPTH - 1:
            x = _conv(x, src, dst, valid, Wu[i], bu[i], n)
        else:
            v = valid.astype(x.dtype)[:, None]
            msg = x[src] * v
            seg = jnp.where(valid, dst, n)
            agg = jax.ops.segment_sum(msg, seg, num_segments=n + 1)[:n]
            x = _final_conv(x + agg, Wu[i], bu[i])
    return x
